# baseline (speedup 1.0000x reference)
import functools

import jax
import jax.numpy as jnp
from jax import lax
from jax.experimental import pallas as pl
from jax.experimental.pallas import tpu as pltpu
from jax.experimental.pallas import tpu_sc as plsc

EMBED = 768
SEQ = 2048
BATCH = 4
NTOK = BATCH * SEQ
NW = 32
POSW = SEQ // NW
LANES = 16
EMB_VECS = EMBED // LANES

_mesh = plsc.VectorSubcoreMesh(core_axis_name="c", subcore_axis_name="s")


@functools.partial(
    pl.kernel,
    out_type=jax.ShapeDtypeStruct((NTOK, EMBED), jnp.float32),
    mesh=_mesh,
    scratch_types=[
        pltpu.VMEM((POSW,), jnp.int32),
        pltpu.VMEM((POSW, EMBED), jnp.float32),
        pltpu.VMEM((POSW, EMBED), jnp.float32),
        pltpu.SemaphoreType.DMA,
    ],
)
def _embed_add(ids_hbm, wte_hbm, wpe_hbm, out_hbm, idx_v, tok_v, pos_v, sem):
    wid = lax.axis_index("s") * 2 + lax.axis_index("c")
    p0 = wid * POSW
    pltpu.sync_copy(wpe_hbm.at[pl.ds(p0, POSW)], pos_v)
    for b in range(BATCH):
        base = b * SEQ + p0
        pltpu.sync_copy(ids_hbm.at[pl.ds(base, POSW)], idx_v)
        pltpu.async_copy(wte_hbm.at[idx_v], tok_v, sem).wait()

        def row_add(r):
            for k in range(EMB_VECS):
                sl = pl.ds(k * LANES, LANES)
                plsc.addupdate(tok_v.at[r, sl], pos_v[r, sl])

        pl.loop(0, POSW, unroll=4)(row_add)
        pltpu.sync_copy(tok_v, out_hbm.at[pl.ds(base, POSW)])


def kernel(input_ids, wte, wpe):
    b, s = input_ids.shape
    ids = input_ids.reshape(-1).astype(jnp.int32)
    out = _embed_add(ids, wte, wpe)
    return out.reshape(b, s, EMBED)
